# baseline (speedup 1.0000x reference)
import functools

import jax
import jax.numpy as jnp
from jax import lax
from jax.experimental import pallas as pl
from jax.experimental.pallas import tpu as pltpu
from jax.experimental.pallas import tpu_sc as plsc

PAD = 0
D_MODEL = 64
NUM_CORES = 2
NUM_SUBCORES = 16
NUM_WORKERS = NUM_CORES * NUM_SUBCORES

IDX_PW = 6400
DESC = 128
K = 2
GROUP = K * DESC
N_GROUPS = IDX_PW // GROUP
NBUF = 5
SNBUF = 2


def _masks_body(tok_ref, pad_ref, seq_ref):
    pad_ref[...] = tok_ref[...] == PAD
    n = seq_ref.shape[0]
    row = lax.broadcasted_iota(jnp.int32, (n, n), 0)
    col = lax.broadcasted_iota(jnp.int32, (n, n), 1)
    seq_ref[...] = col > row


@functools.lru_cache(maxsize=None)
def _make_gather(n_tok):
    assert n_tok == IDX_PW * NUM_WORKERS
    mesh = plsc.VectorSubcoreMesh(core_axis_name="c", subcore_axis_name="s")

    @functools.partial(
        pl.kernel,
        mesh=mesh,
        out_type=jax.ShapeDtypeStruct((n_tok, D_MODEL), jnp.float32),
        scratch_types=[
            pltpu.VMEM((IDX_PW // DESC, DESC), jnp.int32),
            pltpu.VMEM((NBUF, GROUP, D_MODEL), jnp.float32),
            pltpu.VMEM_SHARED((NUM_SUBCORES, SNBUF, GROUP, D_MODEL),
                              jnp.float32),
        ]
        + [pltpu.SemaphoreType.DMA] * (NBUF + SNBUF),
        compiler_params=pltpu.CompilerParams(use_tc_tiling_on_sc=False),
    )
    def k(tok_hbm, table_hbm, out_hbm, idx_v, rows_v, shared, *sems):
        gsem = sems[:NBUF]
        ssem = sems[NBUF:]
        sid = lax.axis_index("s")
        wid = sid * NUM_CORES + lax.axis_index("c")
        base = wid * IDX_PW

        n_desc = IDX_PW // DESC
        pltpu.sync_copy(tok_hbm.at[pl.ds(wid * n_desc, n_desc)], idx_v)

        def issue_gather(g, s):
            for d in range(K):
                pltpu.async_copy(
                    table_hbm.at[idx_v.at[g * K + d]],
                    rows_v.at[s, pl.ds(d * DESC, DESC)], gsem[s])

        def wait_gather(s):
            for d in range(K):
                pltpu.make_async_copy(
                    table_hbm.at[pl.ds(0, DESC)],
                    rows_v.at[s, pl.ds(d * DESC, DESC)], gsem[s]).wait()

        def issue_store(g, s, u, first):
            if not first:
                wait_store(u)
            pltpu.sync_copy(rows_v.at[s], shared.at[sid, u])
            pltpu.async_copy(shared.at[sid, u],
                             out_hbm.at[pl.ds(base + g * GROUP, GROUP)],
                             ssem[u])

        def wait_store(u):
            pltpu.make_async_copy(
                table_hbm.at[pl.ds(0, GROUP)], shared.at[sid, u],
                ssem[u]).wait()

        for g in range(NBUF - 1):
            issue_gather(g, g)

        for g in range(NBUF):
            s = g % NBUF
            t = (g + NBUF - 1) % NBUF
            issue_gather(g + NBUF - 1, t)
            wait_gather(s)
            issue_store(g, s, g % SNBUF, g < SNBUF)

        def body(o, carry):
            for j in range(NBUF):
                g = NBUF * (o + 1) + j
                s = j
                t = (j + NBUF - 1) % NBUF
                issue_gather(g + NBUF - 1, t)
                wait_gather(s)
                issue_store(g, s, j % SNBUF, False)
            return carry

        lax.fori_loop(0, (N_GROUPS - 2 * NBUF) // NBUF, body, 0)

        for j in range(NBUF):
            g = N_GROUPS - NBUF + j
            s = g % NBUF
            if g + NBUF - 1 < N_GROUPS:
                issue_gather(g + NBUF - 1, (g + NBUF - 1) % NBUF)
            wait_gather(s)
            issue_store(g, s, g % SNBUF, False)

        for u in range(SNBUF):
            wait_store(u)

    return k


def kernel(tokens, table):
    bsz, seq_len = tokens.shape
    tok32 = tokens.astype(jnp.int32)
    feats = _make_gather(bsz * seq_len)(tok32.reshape(-1, DESC), table)
    pad, seq = pl.pallas_call(
        _masks_body,
        out_shape=(
            jax.ShapeDtypeStruct((bsz, seq_len), jnp.bool_),
            jax.ShapeDtypeStruct((seq_len, seq_len), jnp.bool_),
        ),
    )(tok32)
    return (feats.reshape(bsz, seq_len, D_MODEL),
            pad[:, None, None, :], seq)
